# Initial kernel scaffold; baseline (speedup 1.0000x reference)
import jax
import jax.numpy as jnp
from jax.experimental import pallas as pl


def kernel(adj_index, adj_values, hv_row, hv_col, hu_row, hu_col, user_emb_w, item_emb_w):
    raise NotImplementedError("write your pallas kernel here")


# speedup vs baseline: 1.8763x; 1.8763x over previous
import functools

import jax
import jax.numpy as jnp
from jax import lax
from jax.experimental import pallas as pl
from jax.experimental.pallas import tpu as pltpu
from jax.experimental.pallas import tpu_sc as plsc

NUM_USERS = 25000
NUM_ITEMS = 25000
DIM = 64
N_LAYERS = 3
N_BICLIQUES = 10000
N_TOTAL = NUM_USERS + NUM_ITEMS

_NC = 2
_NS = 16
_KB = 128


def _pad_to(n, m):
    return ((n + m - 1) // m) * m


def _make_seg_kernel(e_pad, n_half, n_tot, w, has_vals):
    ep = e_pad // _NS
    nb = ep // _KB
    z = n_tot // _NS

    mesh = plsc.VectorSubcoreMesh(core_axis_name="c", subcore_axis_name="s")

    scratch = [
        pltpu.VMEM((_KB,), jnp.int32),
        pltpu.VMEM((_KB,), jnp.int32),
        pltpu.VMEM((_KB,), jnp.int32),
        pltpu.VMEM((_KB, 16), jnp.float32),
        pltpu.VMEM((_KB, w), jnp.float32),
        pltpu.VMEM_SHARED((n_tot, w), jnp.float32),
        pltpu.SemaphoreType.DMA,
    ]

    def _body(col_hbm, row_hbm, val_hbm, tab_hbm, zeros_hbm, out_hbm,
              col_v, row_v, lidx_v, val_v, rows_v, acc, sem):
        c = lax.axis_index("c")
        s = lax.axis_index("s")
        pltpu.sync_copy(zeros_hbm.at[pl.ds(s * z, z)], acc.at[pl.ds(s * z, z)])
        plsc.subcore_barrier()
        cbase = c * n_half
        base0 = s * ep

        def blk(b, carry):
            base = base0 + b * _KB
            pltpu.sync_copy(col_hbm.at[pl.ds(base, _KB)], col_v)
            pltpu.sync_copy(row_hbm.at[pl.ds(base, _KB)], row_v)
            if has_vals:
                pltpu.sync_copy(val_hbm.at[pl.ds(base, _KB)], val_v)
            pltpu.async_copy(tab_hbm.at[col_v], rows_v, sem).wait()
            if has_vals:
                def scale(e, cc):
                    v16 = val_v[e, :]
                    for q in range(w // 16):
                        seg = rows_v[e, pl.ds(q * 16, 16)]
                        rows_v[e, pl.ds(q * 16, 16)] = seg * v16
                    return cc
                lax.fori_loop(0, _KB, scale, 0)
            for t in range(_KB // 16):
                r = row_v[pl.ds(t * 16, 16)]
                loc = r - cbase
                ok = (loc >= 0) & (loc < n_half)
                lidx_v[pl.ds(t * 16, 16)] = jnp.where(ok, loc, n_half)
            pltpu.sync_copy(rows_v, acc.at[lidx_v], add=True)
            return carry

        lax.fori_loop(0, nb, blk, 0)
        plsc.subcore_barrier()
        pltpu.sync_copy(acc.at[pl.ds(s * z, z)],
                        out_hbm.at[pl.ds(c * n_tot + s * z, z)])

    if has_vals:
        def body(col_hbm, row_hbm, val_hbm, tab_hbm, zeros_hbm, out_hbm,
                 col_v, row_v, lidx_v, val_v, rows_v, acc, sem):
            _body(col_hbm, row_hbm, val_hbm, tab_hbm, zeros_hbm, out_hbm,
                  col_v, row_v, lidx_v, val_v, rows_v, acc, sem)
    else:
        def body(col_hbm, row_hbm, tab_hbm, zeros_hbm, out_hbm,
                 col_v, row_v, lidx_v, val_v, rows_v, acc, sem):
            _body(col_hbm, row_hbm, None, tab_hbm, zeros_hbm, out_hbm,
                  col_v, row_v, lidx_v, val_v, rows_v, acc, sem)

    return functools.partial(
        pl.kernel, mesh=mesh,
        out_type=jax.ShapeDtypeStruct((_NC * n_tot, w), jnp.float32),
        scratch_types=scratch,
        compiler_params=pltpu.CompilerParams(use_tc_tiling_on_sc=False),
    )(body)


def _run_seg(kern, cols, rows, vals, tab, n_half, n_tot):
    if vals is None:
        out2 = kern(cols, rows, tab, jnp.zeros((n_tot, tab.shape[1]), jnp.float32))
    else:
        out2 = kern(cols, rows, vals, tab, jnp.zeros((n_tot, tab.shape[1]), jnp.float32))
    return jnp.concatenate([out2[:n_half], out2[n_tot:n_tot + n_half]], axis=0)


def _mean4(a, b, c, d):
    blk = 1000

    def f(a_, b_, c_, d_, o_):
        o_[...] = (a_[...] + b_[...] + c_[...] + d_[...]) * 0.25

    spec = pl.BlockSpec((blk, DIM), lambda i: (i, 0))
    return pl.pallas_call(
        f, grid=(N_TOTAL // blk,),
        in_specs=[spec] * 4, out_specs=spec,
        out_shape=jax.ShapeDtypeStruct((N_TOTAL, DIM), jnp.float32),
    )(a, b, c, d)


def _norm_aug(x, n):
    blk = 1000

    def f(x_, o_):
        xv = x_[...]
        cnt = jnp.maximum(xv[:, DIM:DIM + 1], 1.0)
        o_[...] = jnp.concatenate(
            [xv[:, :DIM] / cnt,
             jnp.ones((blk, 1), jnp.float32),
             jnp.zeros((blk, 15), jnp.float32)], axis=1)

    spec = pl.BlockSpec((blk, 80), lambda i: (i, 0))
    return pl.pallas_call(
        f, grid=(n // blk,),
        in_specs=[spec], out_specs=spec,
        out_shape=jax.ShapeDtypeStruct((n, 80), jnp.float32),
    )(x)


def _final_combine(mean_u, ul):
    blk = 1000

    def f(m_, u_, o_):
        uv = u_[...]
        cnt = jnp.maximum(uv[:, DIM:DIM + 1], 1.0)
        o_[...] = m_[...] + uv[:, :DIM] / cnt

    return pl.pallas_call(
        f, grid=(NUM_USERS // blk,),
        in_specs=[pl.BlockSpec((blk, DIM), lambda i: (i, 0)),
                  pl.BlockSpec((blk, 80), lambda i: (i, 0))],
        out_specs=pl.BlockSpec((blk, DIM), lambda i: (i, 0)),
        out_shape=jax.ShapeDtypeStruct((NUM_USERS, DIM), jnp.float32),
    )(mean_u, ul)


def _pad_edges(rows, cols, vals, e_pad):
    e = rows.shape[0]
    pr = jnp.pad(rows.astype(jnp.int32), (0, e_pad - e), constant_values=-1)
    pc = jnp.pad(cols.astype(jnp.int32), (0, e_pad - e), constant_values=0)
    pv = None
    if vals is not None:
        pv = jnp.pad(vals, (0, e_pad - e), constant_values=0.0)
    return pr, pc, pv


def kernel(adj_index, adj_values, hv_row, hv_col, hu_row, hu_col,
           user_emb_w, item_emb_w):
    e_adj = _pad_to(adj_index.shape[1], _NS * _KB)
    ar, ac, av = _pad_edges(adj_index[0], adj_index[1], adj_values, e_adj)
    av = jnp.broadcast_to(av[:, None], (e_adj, 16))
    n_half_a = N_TOTAL // 2
    n_tot_a = _pad_to(n_half_a + 1, 128)
    spmm_k = _make_seg_kernel(e_adj, n_half_a, n_tot_a, DIM, has_vals=True)

    e0 = jnp.concatenate([user_emb_w, item_emb_w], axis=0)
    e1 = _run_seg(spmm_k, ac, ar, av, e0, n_half_a, n_tot_a)
    e2 = _run_seg(spmm_k, ac, ar, av, e1, n_half_a, n_tot_a)
    e3 = _run_seg(spmm_k, ac, ar, av, e2, n_half_a, n_tot_a)
    final_emb = _mean4(e0, e1, e2, e3)

    tab_i = jnp.concatenate(
        [item_emb_w,
         jnp.ones((NUM_ITEMS, 1), jnp.float32),
         jnp.zeros((NUM_ITEMS, 15), jnp.float32)], axis=1)
    e_hv = _pad_to(hv_row.shape[0], _NS * _KB)
    vr, vc, _ = _pad_edges(hv_row, hv_col, None, e_hv)
    n_half_v = N_BICLIQUES // 2
    n_tot_v = _pad_to(n_half_v + 1, 128)
    hv_k = _make_seg_kernel(e_hv, n_half_v, n_tot_v, 80, has_vals=False)
    bf_raw = _run_seg(hv_k, vc, vr, None, tab_i, n_half_v, n_tot_v)
    bf = _norm_aug(bf_raw, N_BICLIQUES)

    e_hu = _pad_to(hu_row.shape[0], _NS * _KB)
    ur, uc, _ = _pad_edges(hu_row, hu_col, None, e_hu)
    n_half_u = NUM_USERS // 2
    n_tot_u = _pad_to(n_half_u + 1, 128)
    hu_k = _make_seg_kernel(e_hu, n_half_u, n_tot_u, 80, has_vals=False)
    ul = _run_seg(hu_k, uc, ur, None, bf, n_half_u, n_tot_u)

    u_final = _final_combine(final_emb[:NUM_USERS], ul)
    i_global = final_emb[NUM_USERS:]
    return (u_final, i_global)


# speedup vs baseline: 2.3210x; 1.2370x over previous
import functools

import jax
import jax.numpy as jnp
from jax import lax
from jax.experimental import pallas as pl
from jax.experimental.pallas import tpu as pltpu
from jax.experimental.pallas import tpu_sc as plsc

NUM_USERS = 25000
NUM_ITEMS = 25000
DIM = 64
N_LAYERS = 3
N_BICLIQUES = 10000
N_TOTAL = NUM_USERS + NUM_ITEMS

_NC = 2
_NS = 16
_KB = 128


def _pad_to(n, m):
    return ((n + m - 1) // m) * m


def _make_seg_kernel(e_pad, n_half, n_tot, w, has_vals):
    ep = e_pad // _NS
    nb = ep // _KB
    z = n_tot // _NS

    mesh = plsc.VectorSubcoreMesh(core_axis_name="c", subcore_axis_name="s")

    scratch = (
        [pltpu.VMEM((_KB,), jnp.int32)] * 2
        + [pltpu.VMEM((_KB,), jnp.int32)] * 2
        + [pltpu.VMEM((_KB,), jnp.int32)] * 2
        + [pltpu.VMEM((_KB, 16), jnp.float32)] * 2
        + [pltpu.VMEM((_KB, w), jnp.float32)] * 2
        + [pltpu.VMEM_SHARED((n_tot, w), jnp.float32)]
        + [pltpu.SemaphoreType.DMA] * 2
    )

    def _body(col_hbm, row_hbm, val_hbm, tab_hbm, zeros_hbm, out_hbm,
              col0, col1, row0, row1, lidx0, lidx1, val0, val1,
              rows0, rows1, acc, sem0, sem1):
        c = lax.axis_index("c")
        s = lax.axis_index("s")
        pltpu.sync_copy(zeros_hbm.at[pl.ds(s * z, z)], acc.at[pl.ds(s * z, z)])
        plsc.subcore_barrier()
        cbase = c * n_half
        base0 = s * ep

        def fetch(base, col_v, row_v, val_v, rows_v, sem):
            pltpu.sync_copy(col_hbm.at[pl.ds(base, _KB)], col_v)
            pltpu.sync_copy(row_hbm.at[pl.ds(base, _KB)], row_v)
            if has_vals:
                pltpu.sync_copy(val_hbm.at[pl.ds(base, _KB)], val_v)
            return pltpu.async_copy(tab_hbm.at[col_v], rows_v, sem)

        def mklidx(row_v, lidx_v):
            for t in range(_KB // 16):
                r = row_v[pl.ds(t * 16, 16)]
                loc = r - cbase
                ok = (loc >= 0) & (loc < n_half)
                lidx_v[pl.ds(t * 16, 16)] = jnp.where(ok, loc, n_half)

        def drain(cp, rows_v, val_v, lidx_v):
            cp.wait()
            if has_vals:
                def scale(e, cc):
                    v16 = val_v[e, :]
                    for q in range(w // 16):
                        seg = rows_v[e, pl.ds(q * 16, 16)]
                        rows_v[e, pl.ds(q * 16, 16)] = seg * v16
                    return cc
                lax.fori_loop(0, _KB, scale, 0)
            pltpu.sync_copy(rows_v, acc.at[lidx_v], add=True)

        def blk(b2, carry):
            b0 = base0 + (2 * b2) * _KB
            b1 = b0 + _KB
            cp0 = fetch(b0, col0, row0, val0, rows0, sem0)
            cp1 = fetch(b1, col1, row1, val1, rows1, sem1)
            mklidx(row0, lidx0)
            mklidx(row1, lidx1)
            drain(cp0, rows0, val0, lidx0)
            drain(cp1, rows1, val1, lidx1)
            return carry

        lax.fori_loop(0, nb // 2, blk, 0)
        plsc.subcore_barrier()
        pltpu.sync_copy(acc.at[pl.ds(s * z, z)],
                        out_hbm.at[pl.ds(c * n_tot + s * z, z)])

    if has_vals:
        def body(col_hbm, row_hbm, val_hbm, tab_hbm, zeros_hbm, out_hbm,
                 *rest):
            _body(col_hbm, row_hbm, val_hbm, tab_hbm, zeros_hbm, out_hbm,
                  *rest)
    else:
        def body(col_hbm, row_hbm, tab_hbm, zeros_hbm, out_hbm, *rest):
            _body(col_hbm, row_hbm, None, tab_hbm, zeros_hbm, out_hbm, *rest)

    return functools.partial(
        pl.kernel, mesh=mesh,
        out_type=jax.ShapeDtypeStruct((_NC * n_tot, w), jnp.float32),
        scratch_types=scratch,
        compiler_params=pltpu.CompilerParams(use_tc_tiling_on_sc=False),
    )(body)


def _run_seg(kern, cols, rows, vals, tab, n_half, n_tot):
    if vals is None:
        out2 = kern(cols, rows, tab, jnp.zeros((n_tot, tab.shape[1]), jnp.float32))
    else:
        out2 = kern(cols, rows, vals, tab, jnp.zeros((n_tot, tab.shape[1]), jnp.float32))
    return jnp.concatenate([out2[:n_half], out2[n_tot:n_tot + n_half]], axis=0)


def _mean4(a, b, c, d):
    blk = 1000

    def f(a_, b_, c_, d_, o_):
        o_[...] = (a_[...] + b_[...] + c_[...] + d_[...]) * 0.25

    spec = pl.BlockSpec((blk, DIM), lambda i: (i, 0))
    return pl.pallas_call(
        f, grid=(N_TOTAL // blk,),
        in_specs=[spec] * 4, out_specs=spec,
        out_shape=jax.ShapeDtypeStruct((N_TOTAL, DIM), jnp.float32),
    )(a, b, c, d)


def _norm_aug(x, n):
    blk = 1000

    def f(x_, o_):
        xv = x_[...]
        cnt = jnp.maximum(xv[:, DIM:DIM + 1], 1.0)
        o_[...] = jnp.concatenate(
            [xv[:, :DIM] / cnt,
             jnp.ones((blk, 1), jnp.float32),
             jnp.zeros((blk, 15), jnp.float32)], axis=1)

    spec = pl.BlockSpec((blk, 80), lambda i: (i, 0))
    return pl.pallas_call(
        f, grid=(n // blk,),
        in_specs=[spec], out_specs=spec,
        out_shape=jax.ShapeDtypeStruct((n, 80), jnp.float32),
    )(x)


def _final_combine(mean_u, ul):
    blk = 1000

    def f(m_, u_, o_):
        uv = u_[...]
        cnt = jnp.maximum(uv[:, DIM:DIM + 1], 1.0)
        o_[...] = m_[...] + uv[:, :DIM] / cnt

    return pl.pallas_call(
        f, grid=(NUM_USERS // blk,),
        in_specs=[pl.BlockSpec((blk, DIM), lambda i: (i, 0)),
                  pl.BlockSpec((blk, 80), lambda i: (i, 0))],
        out_specs=pl.BlockSpec((blk, DIM), lambda i: (i, 0)),
        out_shape=jax.ShapeDtypeStruct((NUM_USERS, DIM), jnp.float32),
    )(mean_u, ul)


def _pad_edges(rows, cols, vals, e_pad):
    e = rows.shape[0]
    pr = jnp.pad(rows.astype(jnp.int32), (0, e_pad - e), constant_values=-1)
    pc = jnp.pad(cols.astype(jnp.int32), (0, e_pad - e), constant_values=0)
    pv = None
    if vals is not None:
        pv = jnp.pad(vals, (0, e_pad - e), constant_values=0.0)
    return pr, pc, pv


def kernel(adj_index, adj_values, hv_row, hv_col, hu_row, hu_col,
           user_emb_w, item_emb_w):
    e_adj = _pad_to(adj_index.shape[1], 2 * _NS * _KB)
    ar, ac, av = _pad_edges(adj_index[0], adj_index[1], adj_values, e_adj)
    av = jnp.broadcast_to(av[:, None], (e_adj, 16))
    n_half_a = N_TOTAL // 2
    n_tot_a = _pad_to(n_half_a + 1, 128)
    spmm_k = _make_seg_kernel(e_adj, n_half_a, n_tot_a, DIM, has_vals=True)

    e0 = jnp.concatenate([user_emb_w, item_emb_w], axis=0)
    e1 = _run_seg(spmm_k, ac, ar, av, e0, n_half_a, n_tot_a)
    e2 = _run_seg(spmm_k, ac, ar, av, e1, n_half_a, n_tot_a)
    e3 = _run_seg(spmm_k, ac, ar, av, e2, n_half_a, n_tot_a)
    final_emb = _mean4(e0, e1, e2, e3)

    tab_i = jnp.concatenate(
        [item_emb_w,
         jnp.ones((NUM_ITEMS, 1), jnp.float32),
         jnp.zeros((NUM_ITEMS, 15), jnp.float32)], axis=1)
    e_hv = _pad_to(hv_row.shape[0], 2 * _NS * _KB)
    vr, vc, _ = _pad_edges(hv_row, hv_col, None, e_hv)
    n_half_v = N_BICLIQUES // 2
    n_tot_v = _pad_to(n_half_v + 1, 128)
    hv_k = _make_seg_kernel(e_hv, n_half_v, n_tot_v, 80, has_vals=False)
    bf_raw = _run_seg(hv_k, vc, vr, None, tab_i, n_half_v, n_tot_v)
    bf = _norm_aug(bf_raw, N_BICLIQUES)

    e_hu = _pad_to(hu_row.shape[0], 2 * _NS * _KB)
    ur, uc, _ = _pad_edges(hu_row, hu_col, None, e_hu)
    n_half_u = NUM_USERS // 2
    n_tot_u = _pad_to(n_half_u + 1, 128)
    hu_k = _make_seg_kernel(e_hu, n_half_u, n_tot_u, 80, has_vals=False)
    ul = _run_seg(hu_k, uc, ur, None, bf, n_half_u, n_tot_u)

    u_final = _final_combine(final_emb[:NUM_USERS], ul)
    i_global = final_emb[NUM_USERS:]
    return (u_final, i_global)


# speedup vs baseline: 2.3855x; 1.0278x over previous
import functools

import jax
import jax.numpy as jnp
from jax import lax
from jax.experimental import pallas as pl
from jax.experimental.pallas import tpu as pltpu
from jax.experimental.pallas import tpu_sc as plsc

NUM_USERS = 25000
NUM_ITEMS = 25000
DIM = 64
N_LAYERS = 3
N_BICLIQUES = 10000
N_TOTAL = NUM_USERS + NUM_ITEMS

_NC = 2
_NS = 16
_KB = 128


def _pad_to(n, m):
    return ((n + m - 1) // m) * m


def _make_seg_kernel(e_pad, n_half, n_tot, w, has_vals):
    ep = e_pad // _NS
    nb = ep // _KB
    z = n_tot // _NS

    mesh = plsc.VectorSubcoreMesh(core_axis_name="c", subcore_axis_name="s")

    scratch = (
        [pltpu.VMEM((_KB,), jnp.int32)] * 2
        + [pltpu.VMEM((_KB,), jnp.int32)] * 2
        + [pltpu.VMEM((_KB,), jnp.int32)] * 2
        + [pltpu.VMEM((_KB, 16), jnp.float32)] * 2
        + [pltpu.VMEM((_KB, w), jnp.float32)] * 2
        + [pltpu.VMEM_SHARED((n_tot, w), jnp.float32)]
        + [pltpu.SemaphoreType.DMA] * 2
    )

    def _body(col_hbm, row_hbm, val_hbm, tab_hbm, zeros_hbm, out_hbm,
              col0, col1, row0, row1, lidx0, lidx1, val0, val1,
              rows0, rows1, acc, sem0, sem1):
        c = lax.axis_index("c")
        s = lax.axis_index("s")
        pltpu.sync_copy(zeros_hbm.at[pl.ds(s * z, z)], acc.at[pl.ds(s * z, z)])
        plsc.subcore_barrier()
        cbase = c * n_half
        base0 = s * ep

        def fetch(base, col_v, row_v, val_v, rows_v, sem):
            pltpu.sync_copy(col_hbm.at[pl.ds(base, _KB)], col_v)
            pltpu.sync_copy(row_hbm.at[pl.ds(base, _KB)], row_v)
            if has_vals:
                pltpu.sync_copy(val_hbm.at[pl.ds(base, _KB)], val_v)
            return pltpu.async_copy(tab_hbm.at[col_v], rows_v, sem)

        def mklidx(row_v, lidx_v):
            for t in range(_KB // 16):
                r = row_v[pl.ds(t * 16, 16)]
                loc = r - cbase
                ok = (loc >= 0) & (loc < n_half)
                lidx_v[pl.ds(t * 16, 16)] = jnp.where(ok, loc, n_half)

        def drain(cp, rows_v, val_v, lidx_v):
            cp.wait()
            if has_vals:
                def scale(e4, cc):
                    for j in range(4):
                        e = e4 * 4 + j
                        v16 = val_v[e, :]
                        for q in range(w // 16):
                            seg = rows_v[e, pl.ds(q * 16, 16)]
                            rows_v[e, pl.ds(q * 16, 16)] = seg * v16
                    return cc
                lax.fori_loop(0, _KB // 4, scale, 0)
            pltpu.sync_copy(rows_v, acc.at[lidx_v], add=True)

        def blk(b2, carry):
            b0 = base0 + (2 * b2) * _KB
            b1 = b0 + _KB
            cp0 = fetch(b0, col0, row0, val0, rows0, sem0)
            cp1 = fetch(b1, col1, row1, val1, rows1, sem1)
            mklidx(row0, lidx0)
            mklidx(row1, lidx1)
            drain(cp0, rows0, val0, lidx0)
            drain(cp1, rows1, val1, lidx1)
            return carry

        lax.fori_loop(0, nb // 2, blk, 0)
        plsc.subcore_barrier()
        pltpu.sync_copy(acc.at[pl.ds(s * z, z)],
                        out_hbm.at[pl.ds(c * n_tot + s * z, z)])

    if has_vals:
        def body(col_hbm, row_hbm, val_hbm, tab_hbm, zeros_hbm, out_hbm,
                 *rest):
            _body(col_hbm, row_hbm, val_hbm, tab_hbm, zeros_hbm, out_hbm,
                  *rest)
    else:
        def body(col_hbm, row_hbm, tab_hbm, zeros_hbm, out_hbm, *rest):
            _body(col_hbm, row_hbm, None, tab_hbm, zeros_hbm, out_hbm, *rest)

    return functools.partial(
        pl.kernel, mesh=mesh,
        out_type=jax.ShapeDtypeStruct((_NC * n_tot, w), jnp.float32),
        scratch_types=scratch,
        compiler_params=pltpu.CompilerParams(use_tc_tiling_on_sc=False),
    )(body)


def _run_seg(kern, cols, rows, vals, tab, n_half, n_tot):
    if vals is None:
        out2 = kern(cols, rows, tab, jnp.zeros((n_tot, tab.shape[1]), jnp.float32))
    else:
        out2 = kern(cols, rows, vals, tab, jnp.zeros((n_tot, tab.shape[1]), jnp.float32))
    return jnp.concatenate([out2[:n_half], out2[n_tot:n_tot + n_half]], axis=0)


def _mean4(a, b, c, d):
    blk = 1000

    def f(a_, b_, c_, d_, o_):
        o_[...] = (a_[...] + b_[...] + c_[...] + d_[...]) * 0.25

    spec = pl.BlockSpec((blk, DIM), lambda i: (i, 0))
    return pl.pallas_call(
        f, grid=(N_TOTAL // blk,),
        in_specs=[spec] * 4, out_specs=spec,
        out_shape=jax.ShapeDtypeStruct((N_TOTAL, DIM), jnp.float32),
    )(a, b, c, d)


def _norm_aug(x, n):
    blk = 1000

    def f(x_, o_):
        xv = x_[...]
        cnt = jnp.maximum(xv[:, DIM:DIM + 1], 1.0)
        o_[...] = jnp.concatenate(
            [xv[:, :DIM] / cnt,
             jnp.ones((blk, 1), jnp.float32),
             jnp.zeros((blk, 15), jnp.float32)], axis=1)

    spec = pl.BlockSpec((blk, 80), lambda i: (i, 0))
    return pl.pallas_call(
        f, grid=(n // blk,),
        in_specs=[spec], out_specs=spec,
        out_shape=jax.ShapeDtypeStruct((n, 80), jnp.float32),
    )(x)


def _final_combine(mean_u, ul):
    blk = 1000

    def f(m_, u_, o_):
        uv = u_[...]
        cnt = jnp.maximum(uv[:, DIM:DIM + 1], 1.0)
        o_[...] = m_[...] + uv[:, :DIM] / cnt

    return pl.pallas_call(
        f, grid=(NUM_USERS // blk,),
        in_specs=[pl.BlockSpec((blk, DIM), lambda i: (i, 0)),
                  pl.BlockSpec((blk, 80), lambda i: (i, 0))],
        out_specs=pl.BlockSpec((blk, DIM), lambda i: (i, 0)),
        out_shape=jax.ShapeDtypeStruct((NUM_USERS, DIM), jnp.float32),
    )(mean_u, ul)


def _pad_edges(rows, cols, vals, e_pad):
    e = rows.shape[0]
    pr = jnp.pad(rows.astype(jnp.int32), (0, e_pad - e), constant_values=-1)
    pc = jnp.pad(cols.astype(jnp.int32), (0, e_pad - e), constant_values=0)
    pv = None
    if vals is not None:
        pv = jnp.pad(vals, (0, e_pad - e), constant_values=0.0)
    return pr, pc, pv


def kernel(adj_index, adj_values, hv_row, hv_col, hu_row, hu_col,
           user_emb_w, item_emb_w):
    e_adj = _pad_to(adj_index.shape[1], 2 * _NS * _KB)
    ar, ac, av = _pad_edges(adj_index[0], adj_index[1], adj_values, e_adj)
    av = jnp.broadcast_to(av[:, None], (e_adj, 16))
    n_half_a = N_TOTAL // 2
    n_tot_a = _pad_to(n_half_a + 1, 128)
    spmm_k = _make_seg_kernel(e_adj, n_half_a, n_tot_a, DIM, has_vals=True)

    e0 = jnp.concatenate([user_emb_w, item_emb_w], axis=0)
    e1 = _run_seg(spmm_k, ac, ar, av, e0, n_half_a, n_tot_a)
    e2 = _run_seg(spmm_k, ac, ar, av, e1, n_half_a, n_tot_a)
    e3 = _run_seg(spmm_k, ac, ar, av, e2, n_half_a, n_tot_a)
    final_emb = _mean4(e0, e1, e2, e3)

    tab_i = jnp.concatenate(
        [item_emb_w,
         jnp.ones((NUM_ITEMS, 1), jnp.float32),
         jnp.zeros((NUM_ITEMS, 15), jnp.float32)], axis=1)
    e_hv = _pad_to(hv_row.shape[0], 2 * _NS * _KB)
    vr, vc, _ = _pad_edges(hv_row, hv_col, None, e_hv)
    n_half_v = N_BICLIQUES // 2
    n_tot_v = _pad_to(n_half_v + 1, 128)
    hv_k = _make_seg_kernel(e_hv, n_half_v, n_tot_v, 80, has_vals=False)
    bf_raw = _run_seg(hv_k, vc, vr, None, tab_i, n_half_v, n_tot_v)
    bf = _norm_aug(bf_raw, N_BICLIQUES)

    e_hu = _pad_to(hu_row.shape[0], 2 * _NS * _KB)
    ur, uc, _ = _pad_edges(hu_row, hu_col, None, e_hu)
    n_half_u = NUM_USERS // 2
    n_tot_u = _pad_to(n_half_u + 1, 128)
    hu_k = _make_seg_kernel(e_hu, n_half_u, n_tot_u, 80, has_vals=False)
    ul = _run_seg(hu_k, uc, ur, None, bf, n_half_u, n_tot_u)

    u_final = _final_combine(final_emb[:NUM_USERS], ul)
    i_global = final_emb[NUM_USERS:]
    return (u_final, i_global)
